# Initial kernel scaffold; baseline (speedup 1.0000x reference)
import jax
import jax.numpy as jnp
from jax.experimental import pallas as pl


def kernel(token_ids, weights):
    raise NotImplementedError("write your pallas kernel here")


# speedup vs baseline: 2.7579x; 2.7579x over previous
import functools

import jax
import jax.numpy as jnp
from jax import lax
from jax.experimental import pallas as pl
from jax.experimental.pallas import tpu as pltpu
from jax.experimental.pallas import tpu_sc as plsc

B, T = 4096, 50
D = 128
N = B * T
NC, NS = 2, 16
NW = NC * NS
PER_W = N // NW
CHUNK = 128
NCHUNK = PER_W // CHUNK


def _emb_body(idx_hbm, table_hbm, out_hbm, idx_v, rows_v, sem):
    wid = lax.axis_index("s") * NC + lax.axis_index("c")
    base = wid * PER_W

    def body(i, carry):
        off = base + i * CHUNK
        pltpu.sync_copy(idx_hbm.at[pl.ds(off, CHUNK)], idx_v)
        pltpu.async_copy(table_hbm.at[idx_v], rows_v, sem).wait()
        pltpu.sync_copy(rows_v, out_hbm.at[pl.ds(off, CHUNK)])
        return carry

    lax.fori_loop(0, NCHUNK, body, 0)


@functools.partial(jax.jit, static_argnums=())
def _embedding_lookup(flat_ids, weights):
    mesh = plsc.VectorSubcoreMesh(core_axis_name="c", subcore_axis_name="s")
    k = pl.kernel(
        _emb_body,
        mesh=mesh,
        out_type=jax.ShapeDtypeStruct((N, D), jnp.float32),
        scratch_types=[
            pltpu.VMEM((CHUNK,), jnp.int32),
            pltpu.VMEM((CHUNK, D), jnp.float32),
            pltpu.SemaphoreType.DMA,
        ],
    )
    return k(flat_ids, weights)


def kernel(token_ids, weights):
    flat = token_ids.reshape(-1).astype(jnp.int32)
    out = _embedding_lookup(flat, weights)
    return out.reshape(B, T, D)


# speedup vs baseline: 3.3125x; 1.2011x over previous
import functools

import jax
import jax.numpy as jnp
from jax import lax
from jax.experimental import pallas as pl
from jax.experimental.pallas import tpu as pltpu
from jax.experimental.pallas import tpu_sc as plsc

B, T = 4096, 50
D = 128
N = B * T
NC, NS = 2, 16
NW = NC * NS
PER_W = N // NW
CHUNK = 128
NBUF = 5
GROUP = CHUNK * NBUF
NGROUP = PER_W // GROUP


def _emb_body(idx_hbm, table_hbm, out_hbm, idx_v, bufs, sem_g, sem_s):
    wid = lax.axis_index("s") * NC + lax.axis_index("c")
    base = wid * PER_W

    pltpu.sync_copy(idx_hbm.at[pl.ds(base, PER_W)], idx_v)

    def gather(g, b):
        off = g * GROUP + b * CHUNK
        pltpu.async_copy(
            table_hbm.at[idx_v.at[pl.ds(off, CHUNK)]], bufs.at[b], sem_g.at[b])

    def scatter(g, b):
        off = base + g * GROUP + b * CHUNK
        pltpu.async_copy(bufs.at[b], out_hbm.at[pl.ds(off, CHUNK)], sem_s.at[b])

    def wait_g(b):
        pltpu.make_async_copy(table_hbm.at[pl.ds(0, CHUNK)], bufs.at[b],
                              sem_g.at[b]).wait()

    def wait_s(b):
        pltpu.make_async_copy(bufs.at[b], out_hbm.at[pl.ds(0, CHUNK)],
                              sem_s.at[b]).wait()

    for b in range(NBUF):
        gather(0, b)

    def body(g, carry):
        for b in range(NBUF):
            wait_g(b)
            scatter(g, b)
        for b in range(NBUF):
            wait_s(b)
            gather(g + 1, b)
        return carry

    lax.fori_loop(0, NGROUP - 1, body, 0)

    for b in range(NBUF):
        wait_g(b)
        scatter(NGROUP - 1, b)
    for b in range(NBUF):
        wait_s(b)


@jax.jit
def _embedding_lookup(flat_ids, weights):
    mesh = plsc.VectorSubcoreMesh(core_axis_name="c", subcore_axis_name="s")
    k = pl.kernel(
        _emb_body,
        mesh=mesh,
        out_type=jax.ShapeDtypeStruct((N, D), jnp.float32),
        scratch_types=[
            pltpu.VMEM((PER_W,), jnp.int32),
            pltpu.VMEM((NBUF, CHUNK, D), jnp.float32),
            pltpu.SemaphoreType.DMA((NBUF,)),
            pltpu.SemaphoreType.DMA((NBUF,)),
        ],
    )
    return k(flat_ids, weights)


def kernel(token_ids, weights):
    flat = token_ids.reshape(-1).astype(jnp.int32)
    out = _embedding_lookup(flat, weights)
    return out.reshape(B, T, D)


# speedup vs baseline: 5.8822x; 1.7757x over previous
import jax
import jax.numpy as jnp
from jax import lax
from jax.experimental import pallas as pl
from jax.experimental.pallas import tpu as pltpu
from jax.experimental.pallas import tpu_sc as plsc

B, T = 4096, 50
D = 128
TP = 56
NC, NS = 2, 16
NW = NC * NS
BW = B // NW
NBUF = 8


def _emb_body(idx_hbm, table_hbm, out_hbm, idx_v, bufs, sem_g, sem_s):
    wid = lax.axis_index("s") * NC + lax.axis_index("c")
    b0 = wid * BW

    pltpu.sync_copy(idx_hbm.at[pl.ds(b0 * TP, BW * TP)], idx_v)

    def gather(c, j):
        pltpu.async_copy(
            table_hbm.at[idx_v.at[pl.ds(c * TP, T)]], bufs.at[j], sem_g.at[j])

    def scatter(c, j):
        pltpu.async_copy(bufs.at[j], out_hbm.at[b0 + c], sem_s.at[j])

    def wait_g(j):
        pltpu.make_async_copy(out_hbm.at[0], bufs.at[j], sem_g.at[j]).wait()

    def wait_s(j):
        pltpu.make_async_copy(bufs.at[j], out_hbm.at[0], sem_s.at[j]).wait()

    for j in range(NBUF):
        gather(j, j)

    def body(g, carry):
        c = g * NBUF
        for j in range(NBUF):
            wait_g(j)
            scatter(c + j, j)
        for j in range(NBUF):
            wait_s(j)
            gather(c + NBUF + j, j)
        return carry

    lax.fori_loop(0, BW // NBUF - 1, body, 0)

    c = BW - NBUF
    for j in range(NBUF):
        wait_g(j)
        scatter(c + j, j)
    for j in range(NBUF):
        wait_s(j)


def _embedding_lookup(idx_pad, weights):
    mesh = plsc.VectorSubcoreMesh(core_axis_name="c", subcore_axis_name="s")
    k = pl.kernel(
        _emb_body,
        mesh=mesh,
        out_type=jax.ShapeDtypeStruct((B, T, D), jnp.float32),
        scratch_types=[
            pltpu.VMEM((BW * TP,), jnp.int32),
            pltpu.VMEM((NBUF, T, D), jnp.float32),
            pltpu.SemaphoreType.DMA((NBUF,)),
            pltpu.SemaphoreType.DMA((NBUF,)),
        ],
        compiler_params=pltpu.CompilerParams(use_tc_tiling_on_sc=True),
    )
    return k(idx_pad, weights)


def kernel(token_ids, weights):
    ids = token_ids.astype(jnp.int32)
    idx_pad = jnp.pad(ids, ((0, 0), (0, TP - T))).reshape(-1)
    return _embedding_lookup(idx_pad, weights)
